# baseline (speedup 1.0000x reference)
import functools

import jax
import jax.numpy as jnp
from jax.experimental import pallas as pl
from jax.experimental.pallas import tpu as pltpu

_CDT = jnp.bfloat16


def _conv_body(bm, r, m_ref, x_ref, w_ref, out_ref,
               s_scr, inv_scr, z_scr, cache_scr):
    p = pl.program_id(0)
    i = pl.program_id(1)

    @pl.when(p == 0)
    def _():
        blk = m_ref[...]
        s_scr[pl.ds(i * bm, bm)] = jnp.sum(jnp.abs(blk), axis=1)

        @pl.when(i < r)
        def _():
            cache_scr[pl.ds(i * bm, bm), :] = blk.astype(_CDT)

    @pl.when(p == 1)
    def _():
        @pl.when(i == 0)
        def _():
            s = s_scr[...]
            safe = jnp.where(s != 0, s, 1.0)
            inv = jnp.where(s != 0, 1.0 / jnp.sqrt(safe), 0.0)
            inv_scr[...] = inv
            z = jnp.dot(x_ref[...].astype(_CDT), w_ref[...].astype(_CDT),
                        preferred_element_type=jnp.float32)
            z_scr[...] = (inv[:, None] * z).astype(_CDT)

        invm = inv_scr[pl.ds(i * bm, bm)][:, None]

        @pl.when(i < r)
        def _():
            acc = jnp.dot(cache_scr[pl.ds(i * bm, bm), :], z_scr[...],
                          preferred_element_type=jnp.float32)
            out_ref[...] = jnp.maximum(invm * acc, 0.0)

        @pl.when(i >= r)
        def _():
            acc = jnp.dot(m_ref[...].astype(_CDT), z_scr[...],
                          preferred_element_type=jnp.float32)
            out_ref[...] = jnp.maximum(invm * acc, 0.0)


def _conv(m, x, w, bm=256, r=16):
    n = m.shape[0]
    c = x.shape[1]
    nblk = n // bm
    r = min(r, nblk)
    last = min(r, nblk - 1)
    return pl.pallas_call(
        functools.partial(_conv_body, bm, r),
        grid=(2, nblk),
        in_specs=[
            pl.BlockSpec((bm, n),
                         lambda p, i: (jnp.where(p == 0, i,
                                                 jnp.maximum(i, last)), 0)),
            pl.BlockSpec((n, c), lambda p, i: (0, 0)),
            pl.BlockSpec((c, c), lambda p, i: (0, 0)),
        ],
        out_specs=pl.BlockSpec((bm, c),
                               lambda p, i: (jnp.where(p == 0, 0, i), 0)),
        out_shape=jax.ShapeDtypeStruct((n, c), jnp.float32),
        scratch_shapes=[
            pltpu.VMEM((n,), jnp.float32),
            pltpu.VMEM((n,), jnp.float32),
            pltpu.VMEM((n, c), _CDT),
            pltpu.VMEM((r * bm, n), _CDT),
        ],
    )(m, x, w)


def _agg_gn_body(g, bm, eps, m_ref, x_ref, w_ref, add_ref, b_ref,
                 gam_ref, bet_ref, alp_ref, out_ref,
                 z_scr, v_scr, cnt_scr, sum_scr, sq_scr):
    p = pl.program_id(0)
    i = pl.program_id(1)

    @pl.when(p == 0)
    def _():
        @pl.when(i == 0)
        def _():
            z_scr[...] = jnp.dot(x_ref[...].astype(_CDT),
                                 w_ref[...].astype(_CDT),
                                 preferred_element_type=jnp.float32
                                 ).astype(_CDT)
            cnt_scr[...] = jnp.zeros_like(cnt_scr)
            sum_scr[...] = jnp.zeros_like(sum_scr)
            sq_scr[...] = jnp.zeros_like(sq_scr)

        acc = jnp.dot(m_ref[...].astype(_CDT), z_scr[...],
                      preferred_element_type=jnp.float32)
        v = add_ref[...] + acc
        v_scr[pl.ds(i * bm, bm), :] = v
        b = b_ref[...]
        sg = (jax.lax.broadcasted_iota(jnp.int32, (g, bm), 0)
              == b[None, :]).astype(jnp.float32)
        cnt_scr[...] += jnp.sum(sg, axis=1)
        sum_scr[...] += jnp.dot(sg, v, preferred_element_type=jnp.float32)
        sq_scr[...] += jnp.dot(sg, v * v, preferred_element_type=jnp.float32)

    @pl.when(p == 1)
    def _():
        v = v_scr[pl.ds(i * bm, bm), :]
        b = b_ref[...]
        alpha = alp_ref[...]
        cnt = jnp.maximum(cnt_scr[...], 1.0)[:, None]
        mean = sum_scr[...] / cnt
        var = (sq_scr[...] / cnt
               + mean * mean * (alpha * alpha - 2.0 * alpha)[None, :])
        rstd = 1.0 / jnp.sqrt(var + eps)
        st = (b[:, None] == jax.lax.broadcasted_iota(jnp.int32, (bm, g), 1)
              ).astype(jnp.float32)
        xc = v - jnp.dot(st, alpha[None, :] * mean,
                         preferred_element_type=jnp.float32)
        scale = jnp.dot(st, rstd * gam_ref[...][None, :],
                        preferred_element_type=jnp.float32)
        out_ref[...] = xc * scale + bet_ref[...][None, :]


def _agg_gn(m, x, w, add, batch, gamma, beta, alpha, g, eps=1e-5, bm=256):
    n, k = m.shape
    c = x.shape[1]
    nblk = n // bm
    return pl.pallas_call(
        functools.partial(_agg_gn_body, g, bm, eps),
        grid=(2, nblk),
        in_specs=[
            pl.BlockSpec((bm, k),
                         lambda p, i: (jnp.where(p == 0, i, nblk - 1), 0)),
            pl.BlockSpec((k, c), lambda p, i: (0, 0)),
            pl.BlockSpec((c, c), lambda p, i: (0, 0)),
            pl.BlockSpec((bm, c),
                         lambda p, i: (jnp.where(p == 0, i, nblk - 1), 0)),
            pl.BlockSpec((bm,), lambda p, i: (i,)),
            pl.BlockSpec((c,), lambda p, i: (0,)),
            pl.BlockSpec((c,), lambda p, i: (0,)),
            pl.BlockSpec((c,), lambda p, i: (0,)),
        ],
        out_specs=pl.BlockSpec((bm, c),
                               lambda p, i: (jnp.where(p == 0, 0, i), 0)),
        out_shape=jax.ShapeDtypeStruct((n, c), jnp.float32),
        scratch_shapes=[
            pltpu.VMEM((k, c), _CDT),
            pltpu.VMEM((n, c), jnp.float32),
            pltpu.VMEM((g,), jnp.float32),
            pltpu.VMEM((g, c), jnp.float32),
            pltpu.VMEM((g, c), jnp.float32),
        ],
    )(m, x, w, add, batch.astype(jnp.int32), gamma, beta, alpha)


def kernel(x_0, x_1, x_2, hodge_laplacian_0, hodge_laplacian_1,
           hodge_laplacian_2, incidence_1, incidence_2, batch, batch_1, y,
           W0, W1, W2, Wa1, Wa2,
           gn1_gamma, gn1_beta, gn1_alpha, gn2_gamma, gn2_beta, gn2_alpha):
    g = y.shape[0]
    x0b = _conv(hodge_laplacian_0, x_0, W0)
    x1b = _conv(hodge_laplacian_1, x_1, W1)
    x2b = _conv(hodge_laplacian_2, x_2, W2)
    x_2_out = x2b
    x_1_out = _agg_gn(incidence_2, x_2_out, Wa1, x1b, batch_1,
                      gn1_gamma, gn1_beta, gn1_alpha, g)
    x_0_out = _agg_gn(incidence_1, x_1_out, Wa2, x0b, batch,
                      gn2_gamma, gn2_beta, gn2_alpha, g)
    return (x_0_out, x_1_out, x_2_out)


# speedup vs baseline: 1.1804x; 1.1804x over previous
import functools

import jax
import jax.numpy as jnp
from jax.experimental import pallas as pl
from jax.experimental.pallas import tpu as pltpu

_CDT = jnp.bfloat16


def _conv_body(bm, r, m_ref, x_ref, w_ref, out_ref,
               s_scr, inv_scr, z_scr, cache_scr):
    p = pl.program_id(0)
    i = pl.program_id(1)

    @pl.when(p == 0)
    def _():
        blk = m_ref[...]
        s_scr[pl.ds(i * bm, bm)] = jnp.sum(jnp.abs(blk), axis=1)

        @pl.when(i < r)
        def _():
            cache_scr[pl.ds(i * bm, bm), :] = blk.astype(_CDT)

    @pl.when(p == 1)
    def _():
        @pl.when(i == 0)
        def _():
            s = s_scr[...]
            safe = jnp.where(s != 0, s, 1.0)
            inv = jnp.where(s != 0, 1.0 / jnp.sqrt(safe), 0.0)
            inv_scr[...] = inv
            z = jnp.dot(x_ref[...], w_ref[...].astype(_CDT),
                        preferred_element_type=jnp.float32)
            z_scr[...] = (inv[:, None] * z).astype(_CDT)

        invm = inv_scr[pl.ds(i * bm, bm)][:, None]

        @pl.when(i < r)
        def _():
            acc = jnp.dot(cache_scr[pl.ds(i * bm, bm), :], z_scr[...],
                          preferred_element_type=jnp.float32)
            out_ref[...] = jnp.maximum(invm * acc, 0.0)

        @pl.when(i >= r)
        def _():
            acc = jnp.dot(m_ref[...].astype(_CDT), z_scr[...],
                          preferred_element_type=jnp.float32)
            out_ref[...] = jnp.maximum(invm * acc, 0.0)


def _conv(m, x, w, bm=512, r=7):
    n = m.shape[0]
    c = x.shape[1]
    nblk = n // bm
    r = min(r, nblk)
    last = min(r, nblk - 1)
    return pl.pallas_call(
        functools.partial(_conv_body, bm, r),
        grid=(2, nblk),
        in_specs=[
            pl.BlockSpec((bm, n),
                         lambda p, i: (jnp.where(p == 0, i,
                                                 jnp.maximum(i, last)), 0)),
            pl.BlockSpec((n, c), lambda p, i: (0, 0)),
            pl.BlockSpec((c, c), lambda p, i: (0, 0)),
        ],
        out_specs=pl.BlockSpec((bm, c),
                               lambda p, i: (jnp.where(p == 0, 0, i), 0)),
        out_shape=jax.ShapeDtypeStruct((n, c), jnp.float32),
        scratch_shapes=[
            pltpu.VMEM((n,), jnp.float32),
            pltpu.VMEM((n,), jnp.float32),
            pltpu.VMEM((n, c), _CDT),
            pltpu.VMEM((r * bm, n), _CDT),
        ],
    )(m, x.astype(_CDT), w)


def _agg_gn_body(g, bm, eps, m_ref, x_ref, w_ref, add_ref, b_ref,
                 gam_ref, bet_ref, alp_ref, out_ref,
                 z_scr, v_scr, cnt_scr, sum_scr, sq_scr):
    p = pl.program_id(0)
    i = pl.program_id(1)

    @pl.when(p == 0)
    def _():
        @pl.when(i == 0)
        def _():
            z_scr[...] = jnp.dot(x_ref[...].astype(_CDT),
                                 w_ref[...].astype(_CDT),
                                 preferred_element_type=jnp.float32
                                 ).astype(_CDT)
            cnt_scr[...] = jnp.zeros_like(cnt_scr)
            sum_scr[...] = jnp.zeros_like(sum_scr)
            sq_scr[...] = jnp.zeros_like(sq_scr)

        acc = jnp.dot(m_ref[...].astype(_CDT), z_scr[...],
                      preferred_element_type=jnp.float32)
        v = add_ref[...] + acc
        v_scr[pl.ds(i * bm, bm), :] = v
        b = b_ref[...]
        sg = (jax.lax.broadcasted_iota(jnp.int32, (g, bm), 0)
              == b[None, :]).astype(jnp.float32)
        cnt_scr[...] += jnp.sum(sg, axis=1)
        sum_scr[...] += jnp.dot(sg, v, preferred_element_type=jnp.float32)
        sq_scr[...] += jnp.dot(sg, v * v, preferred_element_type=jnp.float32)

    @pl.when(p == 1)
    def _():
        v = v_scr[pl.ds(i * bm, bm), :]
        b = b_ref[...]
        alpha = alp_ref[...]
        cnt = jnp.maximum(cnt_scr[...], 1.0)[:, None]
        mean = sum_scr[...] / cnt
        var = (sq_scr[...] / cnt
               + mean * mean * (alpha * alpha - 2.0 * alpha)[None, :])
        rstd = 1.0 / jnp.sqrt(var + eps)
        st = (b[:, None] == jax.lax.broadcasted_iota(jnp.int32, (bm, g), 1)
              ).astype(jnp.float32)
        xc = v - jnp.dot(st, alpha[None, :] * mean,
                         preferred_element_type=jnp.float32)
        scale = jnp.dot(st, rstd * gam_ref[...][None, :],
                        preferred_element_type=jnp.float32)
        out_ref[...] = xc * scale + bet_ref[...][None, :]


def _agg_gn(m, x, w, add, batch, gamma, beta, alpha, g, eps=1e-5, bm=512):
    n, k = m.shape
    c = x.shape[1]
    nblk = n // bm
    return pl.pallas_call(
        functools.partial(_agg_gn_body, g, bm, eps),
        grid=(2, nblk),
        in_specs=[
            pl.BlockSpec((bm, k),
                         lambda p, i: (jnp.where(p == 0, i, nblk - 1), 0)),
            pl.BlockSpec((k, c), lambda p, i: (0, 0)),
            pl.BlockSpec((c, c), lambda p, i: (0, 0)),
            pl.BlockSpec((bm, c),
                         lambda p, i: (jnp.where(p == 0, i, nblk - 1), 0)),
            pl.BlockSpec((bm,), lambda p, i: (i,)),
            pl.BlockSpec((c,), lambda p, i: (0,)),
            pl.BlockSpec((c,), lambda p, i: (0,)),
            pl.BlockSpec((c,), lambda p, i: (0,)),
        ],
        out_specs=pl.BlockSpec((bm, c),
                               lambda p, i: (jnp.where(p == 0, 0, i), 0)),
        out_shape=jax.ShapeDtypeStruct((n, c), jnp.float32),
        scratch_shapes=[
            pltpu.VMEM((k, c), _CDT),
            pltpu.VMEM((n, c), jnp.float32),
            pltpu.VMEM((g,), jnp.float32),
            pltpu.VMEM((g, c), jnp.float32),
            pltpu.VMEM((g, c), jnp.float32),
        ],
    )(m, x, w, add, batch.astype(jnp.int32), gamma, beta, alpha)


def kernel(x_0, x_1, x_2, hodge_laplacian_0, hodge_laplacian_1,
           hodge_laplacian_2, incidence_1, incidence_2, batch, batch_1, y,
           W0, W1, W2, Wa1, Wa2,
           gn1_gamma, gn1_beta, gn1_alpha, gn2_gamma, gn2_beta, gn2_alpha):
    g = y.shape[0]
    x0b = _conv(hodge_laplacian_0, x_0, W0)
    x1b = _conv(hodge_laplacian_1, x_1, W1)
    x2b = _conv(hodge_laplacian_2, x_2, W2)
    x_2_out = x2b
    x_1_out = _agg_gn(incidence_2, x_2_out, Wa1, x1b, batch_1,
                      gn1_gamma, gn1_beta, gn1_alpha, g)
    x_0_out = _agg_gn(incidence_1, x_1_out, Wa2, x0b, batch,
                      gn2_gamma, gn2_beta, gn2_alpha, g)
    return (x_0_out, x_1_out, x_2_out)


# speedup vs baseline: 1.2018x; 1.0181x over previous
import functools

import jax
import jax.numpy as jnp
from jax.experimental import pallas as pl
from jax.experimental.pallas import tpu as pltpu

_CDT = jnp.bfloat16


def _conv_body(bm, r, emit_bf, m_ref, x_ref, w_ref, *rest):
    if emit_bf:
        out_ref, outbf_ref, s_scr, inv_scr, z_scr, cache_scr = rest
    else:
        out_ref, s_scr, inv_scr, z_scr, cache_scr = rest
    p = pl.program_id(0)
    i = pl.program_id(1)

    @pl.when(p == 0)
    def _():
        blk = m_ref[...]
        s_scr[pl.ds(i * bm, bm)] = jnp.sum(jnp.abs(blk), axis=1)

        @pl.when(i < r)
        def _():
            cache_scr[pl.ds(i * bm, bm), :] = blk.astype(_CDT)

    @pl.when(p == 1)
    def _():
        @pl.when(i == 0)
        def _():
            s = s_scr[...]
            safe = jnp.where(s != 0, s, 1.0)
            inv = jnp.where(s != 0, 1.0 / jnp.sqrt(safe), 0.0)
            inv_scr[...] = inv
            z = jnp.dot(x_ref[...], w_ref[...].astype(_CDT),
                        preferred_element_type=jnp.float32)
            z_scr[...] = (inv[:, None] * z).astype(_CDT)

        invm = inv_scr[pl.ds(i * bm, bm)][:, None]

        @pl.when(i < r)
        def _():
            acc = jnp.dot(cache_scr[pl.ds(i * bm, bm), :], z_scr[...],
                          preferred_element_type=jnp.float32)
            res = jnp.maximum(invm * acc, 0.0)
            out_ref[...] = res
            if emit_bf:
                outbf_ref[...] = res.astype(_CDT)

        @pl.when(i >= r)
        def _():
            acc = jnp.dot(m_ref[...].astype(_CDT), z_scr[...],
                          preferred_element_type=jnp.float32)
            res = jnp.maximum(invm * acc, 0.0)
            out_ref[...] = res
            if emit_bf:
                outbf_ref[...] = res.astype(_CDT)


def _conv(m, x, w, bm=512, r=7, emit_bf=False):
    n = m.shape[0]
    c = x.shape[1]
    nblk = n // bm
    r = min(r, nblk)
    last = min(r, nblk - 1)
    return pl.pallas_call(
        functools.partial(_conv_body, bm, r, emit_bf),
        grid=(2, nblk),
        in_specs=[
            pl.BlockSpec((bm, n),
                         lambda p, i: (jnp.where(p == 0, i,
                                                 jnp.maximum(i, last)), 0)),
            pl.BlockSpec((n, c), lambda p, i: (0, 0)),
            pl.BlockSpec((c, c), lambda p, i: (0, 0)),
        ],
        out_specs=[pl.BlockSpec((bm, c),
                                lambda p, i: (jnp.where(p == 0, 0, i), 0))
                   ] * (2 if emit_bf else 1),
        out_shape=([jax.ShapeDtypeStruct((n, c), jnp.float32)]
                   + ([jax.ShapeDtypeStruct((n, c), _CDT)] if emit_bf else [])),
        scratch_shapes=[
            pltpu.VMEM((n,), jnp.float32),
            pltpu.VMEM((n,), jnp.float32),
            pltpu.VMEM((n, c), _CDT),
            pltpu.VMEM((r * bm, n), _CDT),
        ],
    )(m, x.astype(_CDT), w)


def _agg_gn_body(g, bm, eps, emit_bf, m_ref, x_ref, w_ref, add_ref, b_ref,
                 gam_ref, bet_ref, alp_ref, *rest):
    if emit_bf:
        out_ref, outbf_ref, z_scr, v_scr, cnt_scr, sum_scr, sq_scr = rest
    else:
        out_ref, z_scr, v_scr, cnt_scr, sum_scr, sq_scr = rest
    p = pl.program_id(0)
    i = pl.program_id(1)

    @pl.when(p == 0)
    def _():
        @pl.when(i == 0)
        def _():
            z_scr[...] = jnp.dot(x_ref[...].astype(_CDT),
                                 w_ref[...].astype(_CDT),
                                 preferred_element_type=jnp.float32
                                 ).astype(_CDT)
            cnt_scr[...] = jnp.zeros_like(cnt_scr)
            sum_scr[...] = jnp.zeros_like(sum_scr)
            sq_scr[...] = jnp.zeros_like(sq_scr)

        acc = jnp.dot(m_ref[...].astype(_CDT), z_scr[...],
                      preferred_element_type=jnp.float32)
        v = add_ref[...] + acc
        v_scr[pl.ds(i * bm, bm), :] = v
        b = b_ref[...]
        sg = (jax.lax.broadcasted_iota(jnp.int32, (g, bm), 0)
              == b[None, :]).astype(_CDT)
        cnt_scr[...] += jnp.sum(sg.astype(jnp.float32), axis=1)
        sum_scr[...] += jnp.dot(sg, v.astype(_CDT),
                                preferred_element_type=jnp.float32)
        sq_scr[...] += jnp.dot(sg, (v * v).astype(_CDT),
                               preferred_element_type=jnp.float32)

    @pl.when(p == 1)
    def _():
        v = v_scr[pl.ds(i * bm, bm), :]
        b = b_ref[...]
        alpha = alp_ref[...]
        cnt = jnp.maximum(cnt_scr[...], 1.0)[:, None]
        mean = sum_scr[...] / cnt
        var = (sq_scr[...] / cnt
               + mean * mean * (alpha * alpha - 2.0 * alpha)[None, :])
        rstd = 1.0 / jnp.sqrt(var + eps)
        st = (b[:, None] == jax.lax.broadcasted_iota(jnp.int32, (bm, g), 1)
              ).astype(jnp.float32)
        xc = v - jnp.dot(st, alpha[None, :] * mean,
                         preferred_element_type=jnp.float32)
        scale = jnp.dot(st, rstd * gam_ref[...][None, :],
                        preferred_element_type=jnp.float32)
        res = xc * scale + bet_ref[...][None, :]
        out_ref[...] = res
        if emit_bf:
            outbf_ref[...] = res.astype(_CDT)


def _agg_gn(m, x, w, add, batch, gamma, beta, alpha, g, eps=1e-5, bm=512,
            emit_bf=False):
    n, k = m.shape
    c = x.shape[1]
    nblk = n // bm
    return pl.pallas_call(
        functools.partial(_agg_gn_body, g, bm, eps, emit_bf),
        grid=(2, nblk),
        in_specs=[
            pl.BlockSpec((bm, k),
                         lambda p, i: (jnp.where(p == 0, i, nblk - 1), 0)),
            pl.BlockSpec((k, c), lambda p, i: (0, 0)),
            pl.BlockSpec((c, c), lambda p, i: (0, 0)),
            pl.BlockSpec((bm, c),
                         lambda p, i: (jnp.where(p == 0, i, nblk - 1), 0)),
            pl.BlockSpec((bm,), lambda p, i: (i,)),
            pl.BlockSpec((c,), lambda p, i: (0,)),
            pl.BlockSpec((c,), lambda p, i: (0,)),
            pl.BlockSpec((c,), lambda p, i: (0,)),
        ],
        out_specs=[pl.BlockSpec((bm, c),
                                lambda p, i: (jnp.where(p == 0, 0, i), 0))
                   ] * (2 if emit_bf else 1),
        out_shape=([jax.ShapeDtypeStruct((n, c), jnp.float32)]
                   + ([jax.ShapeDtypeStruct((n, c), _CDT)] if emit_bf else [])),
        scratch_shapes=[
            pltpu.VMEM((k, c), _CDT),
            pltpu.VMEM((n, c), jnp.float32),
            pltpu.VMEM((g,), jnp.float32),
            pltpu.VMEM((g, c), jnp.float32),
            pltpu.VMEM((g, c), jnp.float32),
        ],
    )(m, x, w, add, batch.astype(jnp.int32), gamma, beta, alpha)


def kernel(x_0, x_1, x_2, hodge_laplacian_0, hodge_laplacian_1,
           hodge_laplacian_2, incidence_1, incidence_2, batch, batch_1, y,
           W0, W1, W2, Wa1, Wa2,
           gn1_gamma, gn1_beta, gn1_alpha, gn2_gamma, gn2_beta, gn2_alpha):
    g = y.shape[0]
    (x0b,) = _conv(hodge_laplacian_0, x_0, W0)
    (x1b,) = _conv(hodge_laplacian_1, x_1, W1)
    x2b, x2b_bf = _conv(hodge_laplacian_2, x_2, W2, emit_bf=True)
    x_2_out = x2b
    x_1_out, x1o_bf = _agg_gn(incidence_2, x2b_bf, Wa1, x1b, batch_1,
                              gn1_gamma, gn1_beta, gn1_alpha, g, emit_bf=True)
    (x_0_out,) = _agg_gn(incidence_1, x1o_bf, Wa2, x0b, batch,
                         gn2_gamma, gn2_beta, gn2_alpha, g)
    return (x_0_out, x_1_out, x_2_out)
